# Initial kernel scaffold; baseline (speedup 1.0000x reference)
import jax
import jax.numpy as jnp
from jax.experimental import pallas as pl


def kernel(src, dist, type_dist, env_dist, W_src, b_src, emb_src, W_dist, b_dist, emb_dist, W_env, b_env):
    raise NotImplementedError("write your pallas kernel here")


# speedup vs baseline: 1.1702x; 1.1702x over previous
import functools

import jax
import jax.numpy as jnp
from jax import lax
from jax.experimental import pallas as pl
from jax.experimental.pallas import tpu as pltpu
from jax.experimental.pallas import tpu_sc as plsc

NC, NS = 2, 16
NW = NC * NS
CH = 128


def _pe_table(S, D):
    pos = jnp.arange(S, dtype=jnp.float32)[:, None]
    div = jnp.exp(jnp.arange(0, D, 2, dtype=jnp.float32) * (-jnp.log(10000.0) / D))
    pe = jnp.zeros((S, D), dtype=jnp.float32)
    pe = pe.at[:, 0::2].set(jnp.sin(pos * div))
    pe = pe.at[:, 1::2].set(jnp.cos(pos * div))
    return pe


def _make_sc_kernel(N, NCHW):
    mesh = plsc.VectorSubcoreMesh(
        core_axis_name="c", subcore_axis_name="s", num_cores=NC, num_subcores=NS
    )
    f32 = jnp.float32
    out_sds = jax.ShapeDtypeStruct((N, 128), f32)

    @functools.partial(
        pl.kernel,
        out_type=(out_sds, out_sds, out_sds),
        mesh=mesh,
        scratch_types=[
            pltpu.VMEM((6 * CH,), f32),
            pltpu.VMEM((3 * CH,), jnp.int32),
            pltpu.VMEM((CH, 64), f32),
            pltpu.VMEM((CH, 32), f32),
            pltpu.VMEM((CH, 32), f32),
            pltpu.VMEM((CH, 128), f32),
            pltpu.VMEM((CH, 128), f32),
            pltpu.VMEM((CH, 128), f32),
            pltpu.VMEM((50, 128), f32),
            pltpu.VMEM((50, 128), f32),
            pltpu.VMEM((50, 128), f32),
            pltpu.VMEM((2, 32), f32),
            pltpu.VMEM((2, 32), f32),
            pltpu.VMEM((2, 64), f32),
            pltpu.SemaphoreType.DMA,
            pltpu.SemaphoreType.DMA,
            pltpu.SemaphoreType.DMA,
        ],
        compiler_params=pltpu.CompilerParams(use_tc_tiling_on_sc=False),
    )
    def sc_kernel(
        pf_hbm, pi_hbm, emb_src_hbm, emb_d_hbm,
        bpes_hbm, bped_hbm, bpee_hbm, ws_hbm, wd_hbm, we_hbm,
        o_src_hbm, o_dist_hbm, o_env_hbm,
        pf_v, pi_v, g_src, g_d0, g_d1, o_src, o_dist, o_env,
        bpes_v, bped_v, bpee_v, ws_v, wd_v, we_v,
        sem0, sem1, sem2,
    ):
        w = lax.axis_index("s") * NC + lax.axis_index("c")
        pltpu.sync_copy(bpes_hbm, bpes_v)
        pltpu.sync_copy(bped_hbm, bped_v)
        pltpu.sync_copy(bpee_hbm, bpee_v)
        pltpu.sync_copy(ws_hbm, ws_v)
        pltpu.sync_copy(wd_hbm, wd_v)
        pltpu.sync_copy(we_hbm, we_v)
        ws = [[ws_v[i, 16 * j:16 * (j + 1)] for j in range(2)] for i in range(2)]
        wd = [[wd_v[i, 16 * j:16 * (j + 1)] for j in range(2)] for i in range(2)]
        we = [[we_v[i, 16 * j:16 * (j + 1)] for j in range(4)] for i in range(2)]

        def chunk_body(i, carry):
            c = w * NCHW + i
            base = c * CH
            pltpu.sync_copy(pf_hbm.at[pl.ds(c * 6 * CH, 6 * CH)], pf_v)
            pltpu.sync_copy(pi_hbm.at[pl.ds(c * 3 * CH, 3 * CH)], pi_v)
            cp0 = pltpu.async_copy(emb_src_hbm.at[pi_v.at[pl.ds(0, CH)]], g_src, sem0)
            cp1 = pltpu.async_copy(emb_d_hbm.at[pi_v.at[pl.ds(CH, CH)]], g_d0, sem1)
            cp2 = pltpu.async_copy(emb_d_hbm.at[pi_v.at[pl.ds(2 * CH, CH)]], g_d1, sem2)
            cp0.wait()
            cp1.wait()
            cp2.wait()

            def grp(g, tc):
                tb = 16 * g
                x0v = pf_v[pl.ds(tb, 16)]
                x1v = pf_v[pl.ds(CH + tb, 16)]
                d0v = pf_v[pl.ds(2 * CH + tb, 16)]
                d1v = pf_v[pl.ds(3 * CH + tb, 16)]
                e0v = pf_v[pl.ds(4 * CH + tb, 16)]
                e1v = pf_v[pl.ds(5 * CH + tb, 16)]
                for k in range(16):
                    t = tb + k
                    s = lax.rem(base + t, 50)
                    x0, x1 = x0v[k], x1v[k]
                    d0, d1 = d0v[k], d1v[k]
                    e0, e1 = e0v[k], e1v[k]
                    for j in range(2):
                        a, b = 16 * j, 16 * (j + 1)
                        o_src[t, a:b] = x0 * ws[0][j] + bpes_v[s, a:b]
                        o_src[t, 32 + a:32 + b] = x1 * ws[1][j] + bpes_v[s, 32 + a:32 + b]
                        o_dist[t, a:b] = g_d0[t, a:b] + bped_v[s, a:b]
                        o_dist[t, 32 + a:32 + b] = d0 * wd[0][j] + bped_v[s, 32 + a:32 + b]
                        o_dist[t, 64 + a:64 + b] = g_d1[t, a:b] + bped_v[s, 64 + a:64 + b]
                        o_dist[t, 96 + a:96 + b] = d1 * wd[1][j] + bped_v[s, 96 + a:96 + b]
                    for j in range(4):
                        a, b = 16 * j, 16 * (j + 1)
                        o_src[t, 64 + a:64 + b] = g_src[t, a:b] + bpes_v[s, 64 + a:64 + b]
                        o_env[t, a:b] = e0 * we[0][j] + bpee_v[s, a:b]
                        o_env[t, 64 + a:64 + b] = e1 * we[1][j] + bpee_v[s, 64 + a:64 + b]
                return tc

            lax.fori_loop(0, CH // 16, grp, 0)
            pltpu.sync_copy(o_src, o_src_hbm.at[pl.ds(base, CH)])
            pltpu.sync_copy(o_dist, o_dist_hbm.at[pl.ds(base, CH)])
            pltpu.sync_copy(o_env, o_env_hbm.at[pl.ds(base, CH)])
            return carry

        lax.fori_loop(0, NCHW, chunk_body, 0)

    return sc_kernel


def kernel(src, dist, type_dist, env_dist, W_src, b_src, emb_src,
           W_dist, b_dist, emb_dist, W_env, b_env):
    B, S, _ = src.shape
    N = B * S
    NCH = N // CH
    NCHW = NCH // NW
    V1 = emb_dist.shape[1]

    x0 = src[:, :, 0].reshape(N)
    x1 = src[:, :, 1].reshape(N)
    idx = src[:, :, 2].reshape(N).astype(jnp.int32)
    d0 = dist[:, :, 0].reshape(N)
    d1 = dist[:, :, 1].reshape(N)
    td0 = type_dist[:, :, 0].reshape(N).astype(jnp.int32)
    td1 = type_dist[:, :, 1].reshape(N).astype(jnp.int32) + V1
    e0 = env_dist[:, :, 0].reshape(N)
    e1 = env_dist[:, :, 1].reshape(N)
    pf = (jnp.stack([x0, x1, d0, d1, e0, e1], 0).reshape(6, NCH, CH)
          .transpose(1, 0, 2).reshape(6 * N))
    pi = (jnp.stack([idx, td0, td1], 0).reshape(3, NCH, CH)
          .transpose(1, 0, 2).reshape(3 * N))
    emb_d = emb_dist.reshape(2 * V1, emb_dist.shape[2])

    pe = _pe_table(S, 128)
    bpe_src = jnp.concatenate(
        [b_src[0][None, :] + pe[:, 0:32],
         b_src[1][None, :] + pe[:, 32:64],
         pe[:, 64:128]], axis=1)
    bpe_dist = jnp.concatenate(
        [pe[:, 0:32],
         b_dist[0][None, :] + pe[:, 32:64],
         pe[:, 64:96],
         b_dist[1][None, :] + pe[:, 96:128]], axis=1)
    bpe_env = jnp.concatenate(
        [b_env[0][None, :] + pe[:, 0:64],
         b_env[1][None, :] + pe[:, 64:128]], axis=1)

    sc = _make_sc_kernel(N, NCHW)
    o_src, o_dist, o_env = sc(pf, pi, emb_src, emb_d,
                              bpe_src, bpe_dist, bpe_env, W_src, W_dist, W_env)
    return (o_src.reshape(B, S, 128),
            o_dist.reshape(B, S, 128),
            o_env.reshape(B, S, 128))
